# baseline (speedup 1.0000x reference)
import jax
import jax.numpy as jnp
from jax import lax
from jax.experimental import pallas as pl
from jax.experimental.pallas import tpu as pltpu
from jax.experimental.pallas import tpu_sc as plsc

_CW = 1280
_RPI = 960
_RPW = 480
_G = 16
_NG = _RPW // _G
_LANES = 16


def _sc_body(img_hbm, mask_hbm, rc_hbm, ri_hbm, out_hbm,
             ibuf, mbuf, rc_v, ri_v, i_sem, m_sem, o_sem):
    half = lax.axis_index("c")
    b = lax.axis_index("s")

    pltpu.sync_copy(rc_hbm, rc_v)
    pltpu.sync_copy(ri_hbm, ri_v)
    rc = rc_v[...]
    ri = ri_v[...]
    catf = jnp.where(rc <= 0.001, 0.0, 1.0)
    catf = jnp.where(rc > 0.5, 2.0, catf)
    x = (catf - 1.0) * 10.0 + ri * 10.0
    t = x.astype(jnp.int32)
    idx = t + jnp.where(x > t.astype(jnp.float32), 1, 0)
    idx = jnp.clip(idx, 0, 20)
    lanes = lax.iota(jnp.int32, _LANES)
    bvec = jnp.full((_LANES,), b, jnp.int32)
    i_b_vec = lax.gather(
        idx, bvec[:, None],
        lax.GatherDimensionNumbers(offset_dims=(), collapsed_slice_dims=(0,),
                                   start_index_map=(0,)),
        slice_sizes=(1,), mode=lax.GatherScatterMode.PROMISE_IN_BOUNDS)

    row0 = b * _RPI + half * _RPW
    mrow0_vec = i_b_vec * _RPI + half * _RPW + lanes

    def group(g, _):
        gi = pltpu.async_copy(img_hbm.at[pl.ds(row0 + g * _G, _G)], ibuf, i_sem)
        gm = pltpu.async_copy(mask_hbm.at[mrow0_vec + g * _G], mbuf, m_sem)
        gi.wait()
        gm.wait()

        def row(r, _):
            def col(k, _):
                sl = pl.ds(k * _LANES, _LANES)
                m = mbuf[r, sl]
                im = ibuf[r, sl]
                ibuf[r, sl] = im * (1.0 - m) + m
                return 0
            lax.fori_loop(0, _CW // _LANES, col, 0)
            return 0
        lax.fori_loop(0, _G, row, 0)

        pltpu.async_copy(ibuf, out_hbm.at[pl.ds(row0 + g * _G, _G)], o_sem).wait()
        return 0

    lax.fori_loop(0, _NG, group, 0)


def kernel(img_batch, Mask, rand_category, rand_index):
    B, C, H, W = img_batch.shape
    img2 = img_batch.reshape(B * _RPI, _CW)
    mask2 = Mask.reshape(Mask.shape[0] * _RPI, _CW)
    mesh = plsc.VectorSubcoreMesh(core_axis_name="c", subcore_axis_name="s")
    kfn = pl.kernel(
        _sc_body,
        out_type=jax.ShapeDtypeStruct((B * _RPI, _CW), jnp.float32),
        mesh=mesh,
        scratch_types=[
            pltpu.VMEM((_G, _CW), jnp.float32),
            pltpu.VMEM((_G, _CW), jnp.float32),
            pltpu.VMEM((_LANES,), jnp.float32),
            pltpu.VMEM((_LANES,), jnp.float32),
            pltpu.SemaphoreType.DMA,
            pltpu.SemaphoreType.DMA,
            pltpu.SemaphoreType.DMA,
        ],
    )
    out = kfn(img2, mask2, rand_category, rand_index)
    return out.reshape(B, C, H, W)


# speedup vs baseline: 1.4126x; 1.4126x over previous
import jax
import jax.numpy as jnp
from jax import lax
from jax.experimental import pallas as pl
from jax.experimental.pallas import tpu as pltpu
from jax.experimental.pallas import tpu_sc as plsc

_CW = 1280
_RPI = 960
_RPW = 480
_G = 16
_NG = _RPW // _G
_NGH = _NG // 2
_LANES = 16


def _sc_body(img_hbm, mask_hbm, rc_hbm, ri_hbm, out_hbm,
             ibuf0, ibuf1, mbuf0, mbuf1, obuf0, obuf1, rc_v, ri_v,
             i_sem0, i_sem1, m_sem0, m_sem1, o_sem0, o_sem1):
    half = lax.axis_index("c")
    b = lax.axis_index("s")

    pltpu.sync_copy(rc_hbm, rc_v)
    pltpu.sync_copy(ri_hbm, ri_v)
    rc = rc_v[...]
    ri = ri_v[...]
    catf = jnp.where(rc <= 0.001, 0.0, 1.0)
    catf = jnp.where(rc > 0.5, 2.0, catf)
    x = (catf - 1.0) * 10.0 + ri * 10.0
    t = x.astype(jnp.int32)
    idx = t + jnp.where(x > t.astype(jnp.float32), 1, 0)
    idx = jnp.clip(idx, 0, 20)
    lanes = lax.iota(jnp.int32, _LANES)
    bvec = jnp.full((_LANES,), b, jnp.int32)
    i_b_vec = lax.gather(
        idx, bvec[:, None],
        lax.GatherDimensionNumbers(offset_dims=(), collapsed_slice_dims=(0,),
                                   start_index_map=(0,)),
        slice_sizes=(1,), mode=lax.GatherScatterMode.PROMISE_IN_BOUNDS)

    row0 = b * _RPI + half * _RPW
    mrow0_vec = i_b_vec * _RPI + half * _RPW + lanes

    ibufs = (ibuf0, ibuf1)
    mbufs = (mbuf0, mbuf1)
    obufs = (obuf0, obuf1)
    isems = (i_sem0, i_sem1)
    msems = (m_sem0, m_sem1)
    osems = (o_sem0, o_sem1)

    def issue_in(g, slot):
        pltpu.async_copy(img_hbm.at[pl.ds(row0 + g * _G, _G)],
                         ibufs[slot], isems[slot])
        pltpu.async_copy(mask_hbm.at[mrow0_vec + g * _G],
                         mbufs[slot], msems[slot])

    issue_in(0, 0)
    issue_in(1, 1)

    def halfstep(it, slot):
        g = 2 * it + slot

        @pl.when(it > 0)
        def _():
            pltpu.make_async_copy(obufs[slot], out_hbm.at[pl.ds(0, _G)],
                                  osems[slot]).wait()

        pltpu.make_async_copy(img_hbm.at[pl.ds(0, _G)],
                              ibufs[slot], isems[slot]).wait()
        pltpu.make_async_copy(mask_hbm.at[mrow0_vec],
                              mbufs[slot], msems[slot]).wait()

        ib, mb, ob = ibufs[slot], mbufs[slot], obufs[slot]

        def row(r, _):
            for k in range(_CW // _LANES):
                sl = pl.ds(k * _LANES, _LANES)
                m = mb[r, sl]
                ob[r, sl] = ib[r, sl] * (1.0 - m) + m
            return 0
        lax.fori_loop(0, _G, row, 0)

        pltpu.async_copy(obufs[slot], out_hbm.at[pl.ds(row0 + g * _G, _G)],
                         osems[slot])

        @pl.when(it < _NGH - 1)
        def _():
            issue_in(g + 2, slot)

    def body(it, _):
        halfstep(it, 0)
        halfstep(it, 1)
        return 0
    lax.fori_loop(0, _NGH, body, 0)

    for slot in (0, 1):
        pltpu.make_async_copy(obufs[slot], out_hbm.at[pl.ds(0, _G)],
                              osems[slot]).wait()


def kernel(img_batch, Mask, rand_category, rand_index):
    B, C, H, W = img_batch.shape
    img2 = img_batch.reshape(B * _RPI, _CW)
    mask2 = Mask.reshape(Mask.shape[0] * _RPI, _CW)
    mesh = plsc.VectorSubcoreMesh(core_axis_name="c", subcore_axis_name="s")
    kfn = pl.kernel(
        _sc_body,
        out_type=jax.ShapeDtypeStruct((B * _RPI, _CW), jnp.float32),
        mesh=mesh,
        scratch_types=[
            pltpu.VMEM((_G, _CW), jnp.float32),
            pltpu.VMEM((_G, _CW), jnp.float32),
            pltpu.VMEM((_G, _CW), jnp.float32),
            pltpu.VMEM((_G, _CW), jnp.float32),
            pltpu.VMEM((_G, _CW), jnp.float32),
            pltpu.VMEM((_G, _CW), jnp.float32),
            pltpu.VMEM((_LANES,), jnp.float32),
            pltpu.VMEM((_LANES,), jnp.float32),
            pltpu.SemaphoreType.DMA,
            pltpu.SemaphoreType.DMA,
            pltpu.SemaphoreType.DMA,
            pltpu.SemaphoreType.DMA,
            pltpu.SemaphoreType.DMA,
            pltpu.SemaphoreType.DMA,
        ],
    )
    out = kfn(img2, mask2, rand_category, rand_index)
    return out.reshape(B, C, H, W)
